# baseline (speedup 1.0000x reference)
import functools

import jax
import jax.numpy as jnp
from jax import lax
from jax.experimental import pallas as pl
from jax.experimental.pallas import tpu as pltpu
from jax.experimental.pallas import tpu_sc as plsc

B, T, D = 16, 2048, 64
H = 8
NBUCK = 32
BS = 64
C = H * NBUCK
NT = H * T
NW = 32
TPW = (B * H) // NW
SELF_VAL = -50000.0
GCH = 256
DP = 2 * D


def _hash_body(qk_ref, v_ref, rot_ref, buck_ref, qkv_ref, nrm_ref):
    x = qk_ref[0]
    rT = lax.dot_general(rot_ref[...], x, (((0,), (1,)), ((), ())),
                         preferred_element_type=jnp.float32)
    iota32 = lax.broadcasted_iota(jnp.int32, (NBUCK, T), 0)
    hrows = []
    for h in range(H):
        sub = rT[h * 16:(h + 1) * 16]
        seg = jnp.concatenate([sub, -sub], axis=0)
        m = jnp.max(seg, axis=0, keepdims=True)
        am = jnp.min(jnp.where(seg == m, iota32, NBUCK), axis=0, keepdims=True)
        blocks = [am[:, k * 128:(k + 1) * 128] for k in range(T // 128)]
        hrows.append(jnp.concatenate(blocks, axis=0).reshape(1, T // 128, 128))
    buck_ref[0] = jnp.concatenate(hrows, axis=0)
    xsq = x * x
    n_row = jnp.sum(xsq, axis=1, keepdims=True)
    qkn = x * (1.0 / jnp.maximum(jnp.sqrt(n_row), 1e-12))
    qkv_ref[0] = jnp.concatenate([qkn, v_ref[0]], axis=1)
    n_lane = jnp.sqrt(lax.dot_general(
        jnp.ones((1, D), jnp.float32), xsq, (((1,), (1,)), ((), ())),
        preferred_element_type=jnp.float32))
    nbl = [n_lane[:, k * 128:(k + 1) * 128] for k in range(T // 128)]
    nrm_ref[0] = jnp.concatenate(nbl, axis=0)


def _hash_call(qk, v, rot2):
    return pl.pallas_call(
        _hash_body,
        grid=(B,),
        in_specs=[
            pl.BlockSpec((1, T, D), lambda b: (b, 0, 0)),
            pl.BlockSpec((1, T, D), lambda b: (b, 0, 0)),
            pl.BlockSpec((D, H * 16), lambda b: (0, 0)),
        ],
        out_specs=[
            pl.BlockSpec((1, H, T // 128, 128), lambda b: (b, 0, 0, 0)),
            pl.BlockSpec((1, T, DP), lambda b: (b, 0, 0)),
            pl.BlockSpec((1, T // 128, 128), lambda b: (b, 0, 0)),
        ],
        out_shape=[
            jax.ShapeDtypeStruct((B, H, T // 128, 128), jnp.int32),
            jax.ShapeDtypeStruct((B, T, DP), jnp.float32),
            jax.ShapeDtypeStruct((B, T // 128, 128), jnp.float32),
        ],
    )(qk, v, rot2)


def _sort_gather_kernel(buck_hbm, qkv_hbm, nrm_hbm,
                        st_hbm, pos_hbm, sqkv_hbm, nst_hbm,
                        buk_v, rank_v, st_v, idxg_v, pos_v,
                        cnt_v, bs_v, nrm_v, nst_v, rows_a, rows_b, sem_a, sem_b):
    wid = lax.axis_index("c") * 16 + lax.axis_index("s")
    idx16 = lax.iota(jnp.int32, 16)
    zeros16 = jnp.zeros((16,), jnp.int32)

    def task_body(i, _):
        tid = wid * TPW + i
        b = tid // H
        h = tid % H
        pltpu.sync_copy(buck_hbm.at[b, h], buk_v)
        pltpu.sync_copy(nrm_hbm.at[b], nrm_v)
        cnt_v[pl.ds(0, 16)] = zeros16
        cnt_v[pl.ds(16, 16)] = zeros16

        def blk1(blk, _c):
            bvec = buk_v[blk // 8, pl.ds((blk % 8) * 16, 16)]
            key = bvec * 16 + idx16
            ks, vs = plsc.sort_key_val(key, idx16)
            bs = lax.shift_right_logical(ks, 4)
            bs_v[...] = bs
            prev = plsc.load_gather(bs_v, [jnp.maximum(idx16 - 1, 0)])
            is_start = (idx16 == 0) | (bs != prev)
            start_idx = plsc.cummax(jnp.where(is_start, idx16, 0))
            rnk = (idx16 - start_idx) + plsc.load_gather(cnt_v, [bs])
            nxt = plsc.load_gather(bs_v, [jnp.minimum(idx16 + 1, 15)])
            is_end = (idx16 == 15) | (bs != nxt)
            plsc.store_scatter(cnt_v, [bs], rnk + 1, mask=is_end)
            plsc.store_scatter(rank_v, [blk * 16 + vs], rnk)
            return _c

        lax.fori_loop(0, T // 16, blk1, 0)

        c0 = cnt_v[pl.ds(0, 16)]
        c1 = cnt_v[pl.ds(16, 16)]
        t0 = jnp.sum(c0)
        base0 = plsc.cumsum(c0) - c0
        base1 = plsc.cumsum(c1) + t0 - c1
        cnt_v[pl.ds(0, 16)] = base0
        cnt_v[pl.ds(16, 16)] = base1

        def blk2(blk, _c):
            r = blk // 8
            csl = pl.ds((blk % 8) * 16, 16)
            bvec = buk_v[r, csl]
            rnk = rank_v[pl.ds(blk * 16, 16)]
            ploc = plsc.load_gather(cnt_v, [bvec]) + rnk
            pr = lax.shift_right_logical(ploc, 7)
            pc = lax.bitwise_and(ploc, 127)
            tvec = blk * 16 + idx16
            pos_v[r, csl] = ploc + h * T
            plsc.store_scatter(st_v, [pr, pc], tvec)
            plsc.store_scatter(idxg_v, [ploc], tvec + b * T)
            plsc.store_scatter(nst_v, [pr, pc], nrm_v[r, csl])
            return _c

        lax.fori_loop(0, T // 16, blk2, 0)

        pltpu.sync_copy(pos_v, pos_hbm.at[b, h])
        pltpu.sync_copy(st_v, st_hbm.at[b, pl.ds(h * (T // 128), T // 128)])
        pltpu.sync_copy(nst_v, nst_hbm.at[b, pl.ds(h * (T // 128), T // 128)])

        NCH = T // GCH
        bufs = (rows_a, rows_b)
        sems = (sem_a, sem_b)

        def fire(cch):
            idx_sl = idxg_v.at[pl.ds(cch * GCH, GCH)]
            return pltpu.async_copy(qkv_hbm.at[idx_sl], bufs[cch % 2], sems[cch % 2])

        cps = [fire(0), fire(1)]
        for cch in range(NCH):
            cps[cch].wait()
            pltpu.sync_copy(bufs[cch % 2],
                            sqkv_hbm.at[b, pl.ds(h * T + cch * GCH, GCH)])
            if cch + 2 < NCH:
                cps.append(fire(cch + 2))
        return _

    lax.fori_loop(0, TPW, task_body, 0)


def _sort_gather_call(buck, qkv_flat, nrm_flat):
    mesh = plsc.VectorSubcoreMesh(core_axis_name="c", subcore_axis_name="s")
    fn = functools.partial(
        pl.kernel,
        mesh=mesh,
        compiler_params=pltpu.CompilerParams(needs_layout_passes=False),
        out_type=[
            jax.ShapeDtypeStruct((B, NT // 128, 128), jnp.int32),
            jax.ShapeDtypeStruct((B, H, T // 128, 128), jnp.int32),
            jax.ShapeDtypeStruct((B, NT, DP), jnp.float32),
            jax.ShapeDtypeStruct((B, NT // 128, 128), jnp.float32),
        ],
        scratch_types=[
            pltpu.VMEM((T // 128, 128), jnp.int32),
            pltpu.VMEM((T,), jnp.int32),
            pltpu.VMEM((T // 128, 128), jnp.int32),
            pltpu.VMEM((T,), jnp.int32),
            pltpu.VMEM((T // 128, 128), jnp.int32),
            pltpu.VMEM((32,), jnp.int32),
            pltpu.VMEM((16,), jnp.int32),
            pltpu.VMEM((T // 128, 128), jnp.float32),
            pltpu.VMEM((T // 128, 128), jnp.float32),
            pltpu.VMEM((GCH, DP), jnp.float32),
            pltpu.VMEM((GCH, DP), jnp.float32),
            pltpu.SemaphoreType.DMA,
            pltpu.SemaphoreType.DMA,
        ],
    )(_sort_gather_kernel)
    return fn(buck, qkv_flat, nrm_flat)


ACH = 16
AR = ACH * BS


def _attn_body(qc, qp1, tq, nq, tkc, tkp, so_ref):
    mv = -jnp.finfo(jnp.float32).max
    qs = [qc[0, j * BS:(j + 1) * BS, :D] for j in range(ACH)]
    vs = [qc[0, j * BS:(j + 1) * BS, D:] for j in range(ACH)]
    qprev = qp1[0, :, :D]
    vprev = qp1[0, :, D:]
    dots_list = []
    for j in range(ACH):
        kmat = jnp.concatenate([qs[j], qprev if j == 0 else qs[j - 1]], axis=0)
        dots_list.append(
            lax.dot_general(qs[j], kmat, (((1,), (1,)), ((), ())),
                            preferred_element_type=jnp.float32))
    dots = jnp.concatenate(dots_list, axis=0)
    dots = dots * (nq[0] * 0.125)

    tcur = tkc[0, :, 0]
    tshift = jnp.concatenate([tkp[0, 0], tcur[:-1]], axis=0)
    ktm = jnp.concatenate([tcur, tshift], axis=1)
    ktb = jnp.broadcast_to(ktm[:, None, :], (ACH, BS, 2 * BS)).reshape(AR, 2 * BS)
    qt = tq[0]
    dots = jnp.where(qt < ktb, mv, dots)
    dots = jnp.where(qt == ktb, SELF_VAL, dots)

    mx = jnp.max(dots, axis=1, keepdims=True)
    ex = jnp.exp(dots - mx)
    s = jnp.sum(ex, axis=1, keepdims=True)
    lse = jnp.log(s) + mx

    o_list = []
    for j in range(ACH):
        vmat = jnp.concatenate([vs[j], vprev if j == 0 else vs[j - 1]], axis=0)
        o_list.append(
            lax.dot_general(ex[j * BS:(j + 1) * BS], vmat,
                            (((1,), (0,)), ((), ())),
                            preferred_element_type=jnp.float32))
    ocat = jnp.concatenate(o_list, axis=0) / s
    so_ref[0] = jnp.concatenate(
        [ocat, jnp.broadcast_to(lse, (AR, D))], axis=1)


def _attn_call(sqkv, tq, nq, tk):
    NI = NT // AR
    NB64 = NT // BS
    cur = pl.BlockSpec((1, AR, DP), lambda b, i: (b, i, 0))
    prev = pl.BlockSpec((1, BS, DP),
                        lambda b, i: (b, (i * ACH + NB64 - 1) % NB64, 0))
    tqs = pl.BlockSpec((1, AR, 1), lambda b, i: (b, i, 0))
    tkc = pl.BlockSpec((1, ACH, 1, BS), lambda b, i: (b, i, 0, 0))
    tkp = pl.BlockSpec((1, 1, 1, BS),
                       lambda b, i: (b, (i * ACH + C - 1) % C, 0, 0))
    return pl.pallas_call(
        _attn_body,
        grid=(B, NI),
        in_specs=[cur, prev, tqs, tqs, tkc, tkp],
        out_specs=pl.BlockSpec((1, AR, DP), lambda b, i: (b, i, 0)),
        out_shape=jax.ShapeDtypeStruct((B, NT, DP), jnp.float32),
    )(sqkv, sqkv, tq, nq, tk, tk)


def _unsort_kernel(pos_hbm, so_hbm,
                   ou_hbm,
                   pos_v, idxg_v, rows_a, rows_b, sem_a, sem_b):
    wid = lax.axis_index("c") * 16 + lax.axis_index("s")
    idx16 = lax.iota(jnp.int32, 16)

    def task_body(i, _):
        tid = wid * TPW + i
        b = tid // H
        h = tid % H
        pltpu.sync_copy(pos_hbm.at[b, h], pos_v)

        def blk(blk_i, _c):
            p = pos_v[blk_i // 8, pl.ds((blk_i % 8) * 16, 16)]
            idxg_v[pl.ds(blk_i * 16, 16)] = p + b * NT
            return _c

        lax.fori_loop(0, T // 16, blk, 0)
        NCH = T // GCH
        bufs = (rows_a, rows_b)
        sems = (sem_a, sem_b)

        def fire(cch):
            idx_sl = idxg_v.at[pl.ds(cch * GCH, GCH)]
            return pltpu.async_copy(so_hbm.at[idx_sl], bufs[cch % 2], sems[cch % 2])

        cps = [fire(0), fire(1)]
        for cch in range(NCH):
            cps[cch].wait()
            pltpu.sync_copy(bufs[cch % 2],
                            ou_hbm.at[b, h, pl.ds(cch * GCH, GCH)])
            if cch + 2 < NCH:
                cps.append(fire(cch + 2))
        return _

    lax.fori_loop(0, TPW, task_body, 0)


def _unsort_call(pos, so_flat):
    mesh = plsc.VectorSubcoreMesh(core_axis_name="c", subcore_axis_name="s")
    fn = functools.partial(
        pl.kernel,
        mesh=mesh,
        compiler_params=pltpu.CompilerParams(needs_layout_passes=False),
        out_type=jax.ShapeDtypeStruct((B, H, T, DP), jnp.float32),
        scratch_types=[
            pltpu.VMEM((T // 128, 128), jnp.int32),
            pltpu.VMEM((T,), jnp.int32),
            pltpu.VMEM((GCH, DP), jnp.float32),
            pltpu.VMEM((GCH, DP), jnp.float32),
            pltpu.SemaphoreType.DMA,
            pltpu.SemaphoreType.DMA,
        ],
    )(_unsort_kernel)
    return fn(pos, so_flat)


def _combine_body(o_ref, out_ref):
    l = o_ref[0, :, :, D:D + 1]
    mx = jnp.max(l, axis=0, keepdims=True)
    w = jnp.exp(l - mx)
    w = w / jnp.sum(w, axis=0, keepdims=True)
    acc = o_ref[0, 0, :, :D] * w[0]
    for h in range(1, H):
        acc = acc + o_ref[0, h, :, :D] * w[h]
    out_ref[0] = acc


def _combine_call(o_uns):
    return pl.pallas_call(
        _combine_body,
        grid=(B,),
        in_specs=[pl.BlockSpec((1, H, T, DP), lambda b: (b, 0, 0, 0))],
        out_specs=pl.BlockSpec((1, T, D), lambda b: (b, 0, 0)),
        out_shape=jax.ShapeDtypeStruct((B, T, D), jnp.float32),
    )(o_uns)


def kernel(qk, v, rotations):
    rot2 = rotations[0].reshape(D, H * 16)
    buck4, qkv, nrm = _hash_call(qk, v, rot2)
    st, pos, sqkv, nst = _sort_gather_call(buck4, qkv.reshape(B * T, DP), nrm)
    stf = st.reshape(B, NT).astype(jnp.float32)
    tq = stf.reshape(B, NT, 1)
    nq = nst.reshape(B, NT, 1)
    tk = stf.reshape(B, C, 1, BS)
    so = _attn_call(sqkv, tq, nq, tk)
    o_uns = _unsort_call(pos, so.reshape(B * NT, DP))
    return _combine_call(o_uns)


# speedup vs baseline: 1.4357x; 1.4357x over previous
import functools

import jax
import jax.numpy as jnp
from jax import lax
from jax.experimental import pallas as pl
from jax.experimental.pallas import tpu as pltpu
from jax.experimental.pallas import tpu_sc as plsc

B, T, D = 16, 2048, 64
H = 8
NBUCK = 32
BS = 64
C = H * NBUCK
NT = H * T
NW = 32
TPW = (B * H) // NW
SELF_VAL = -50000.0
GCH = 256
DP = 2 * D


def _hash_body(qk_ref, v_ref, rot_ref, buck_ref, qkv_ref, nrm_ref):
    x = qk_ref[0]
    rT = lax.dot_general(rot_ref[...], x, (((0,), (1,)), ((), ())),
                         preferred_element_type=jnp.float32)
    iota32 = lax.broadcasted_iota(jnp.int32, (NBUCK, T), 0)
    hrows = []
    for h in range(H):
        sub = rT[h * 16:(h + 1) * 16]
        seg = jnp.concatenate([sub, -sub], axis=0)
        m = jnp.max(seg, axis=0, keepdims=True)
        am = jnp.min(jnp.where(seg == m, iota32, NBUCK), axis=0, keepdims=True)
        blocks = [am[:, k * 128:(k + 1) * 128] for k in range(T // 128)]
        hrows.append(jnp.concatenate(blocks, axis=0).reshape(1, T // 128, 128))
    buck_ref[0] = jnp.concatenate(hrows, axis=0)
    xsq = x * x
    n_row = jnp.sum(xsq, axis=1, keepdims=True)
    qkn = x * (1.0 / jnp.maximum(jnp.sqrt(n_row), 1e-12))
    qkv_ref[0] = jnp.concatenate([qkn, v_ref[0]], axis=1)
    n_lane = jnp.sqrt(lax.dot_general(
        jnp.ones((1, D), jnp.float32), xsq, (((1,), (1,)), ((), ())),
        preferred_element_type=jnp.float32))
    nbl = [n_lane[:, k * 128:(k + 1) * 128] for k in range(T // 128)]
    nrm_ref[0] = jnp.concatenate(nbl, axis=0)


def _hash_call(qk, v, rot2):
    return pl.pallas_call(
        _hash_body,
        grid=(B,),
        in_specs=[
            pl.BlockSpec((1, T, D), lambda b: (b, 0, 0)),
            pl.BlockSpec((1, T, D), lambda b: (b, 0, 0)),
            pl.BlockSpec((D, H * 16), lambda b: (0, 0)),
        ],
        out_specs=[
            pl.BlockSpec((1, H, T // 128, 128), lambda b: (b, 0, 0, 0)),
            pl.BlockSpec((1, T, DP), lambda b: (b, 0, 0)),
            pl.BlockSpec((1, T // 128, 128), lambda b: (b, 0, 0)),
        ],
        out_shape=[
            jax.ShapeDtypeStruct((B, H, T // 128, 128), jnp.int32),
            jax.ShapeDtypeStruct((B, T, DP), jnp.float32),
            jax.ShapeDtypeStruct((B, T // 128, 128), jnp.float32),
        ],
    )(qk, v, rot2)


def _sort_gather_kernel(buck_hbm, qkv_hbm, nrm_hbm,
                        st_hbm, pos_hbm, sqkv_hbm, nst_hbm,
                        buk_v, rank_v, st_v, idxg_v, pos_v,
                        cnt_v, bs_v, nrm_v, nst_v, rows_a, rows_b, sem_a, sem_b):
    wid = lax.axis_index("c") * 16 + lax.axis_index("s")
    idx16 = lax.iota(jnp.int32, 16)
    zeros16 = jnp.zeros((16,), jnp.int32)

    def task_body(i, _):
        tid = wid * TPW + i
        b = tid // H
        h = tid % H
        pltpu.sync_copy(buck_hbm.at[b, h], buk_v)
        pltpu.sync_copy(nrm_hbm.at[b], nrm_v)
        cnt_v[pl.ds(0, 16)] = zeros16
        cnt_v[pl.ds(16, 16)] = zeros16

        def blk1(blk, _c):
            bvec = buk_v[blk // 8, pl.ds((blk % 8) * 16, 16)]
            key = bvec * 16 + idx16
            ks, vs = plsc.sort_key_val(key, idx16)
            bs = lax.shift_right_logical(ks, 4)
            bs_v[...] = bs
            prev = plsc.load_gather(bs_v, [jnp.maximum(idx16 - 1, 0)])
            is_start = (idx16 == 0) | (bs != prev)
            start_idx = plsc.cummax(jnp.where(is_start, idx16, 0))
            rnk = (idx16 - start_idx) + plsc.load_gather(cnt_v, [bs])
            nxt = plsc.load_gather(bs_v, [jnp.minimum(idx16 + 1, 15)])
            is_end = (idx16 == 15) | (bs != nxt)
            plsc.store_scatter(cnt_v, [bs], rnk + 1, mask=is_end)
            plsc.store_scatter(rank_v, [blk * 16 + vs], rnk)
            return _c

        lax.fori_loop(0, T // 16, blk1, 0)

        c0 = cnt_v[pl.ds(0, 16)]
        c1 = cnt_v[pl.ds(16, 16)]
        t0 = jnp.sum(c0)
        base0 = plsc.cumsum(c0) - c0
        base1 = plsc.cumsum(c1) + t0 - c1
        cnt_v[pl.ds(0, 16)] = base0
        cnt_v[pl.ds(16, 16)] = base1

        def blk2(blk, _c):
            r = blk // 8
            csl = pl.ds((blk % 8) * 16, 16)
            bvec = buk_v[r, csl]
            rnk = rank_v[pl.ds(blk * 16, 16)]
            ploc = plsc.load_gather(cnt_v, [bvec]) + rnk
            pr = lax.shift_right_logical(ploc, 7)
            pc = lax.bitwise_and(ploc, 127)
            tvec = blk * 16 + idx16
            pos_v[r, csl] = ploc + h * T
            plsc.store_scatter(st_v, [pr, pc], tvec)
            plsc.store_scatter(idxg_v, [ploc], tvec + b * T)
            plsc.store_scatter(nst_v, [pr, pc], nrm_v[r, csl])
            return _c

        lax.fori_loop(0, T // 16, blk2, 0)

        pltpu.sync_copy(pos_v, pos_hbm.at[b, h])
        pltpu.sync_copy(st_v, st_hbm.at[b, pl.ds(h * (T // 128), T // 128)])
        pltpu.sync_copy(nst_v, nst_hbm.at[b, pl.ds(h * (T // 128), T // 128)])

        NCH = T // GCH
        bufs = (rows_a, rows_b)
        sems = (sem_a, sem_b)

        def fire(cch):
            idx_sl = idxg_v.at[pl.ds(cch * GCH, GCH)]
            return pltpu.async_copy(qkv_hbm.at[idx_sl], bufs[cch % 2], sems[cch % 2])

        cps = [fire(0), fire(1)]
        for cch in range(NCH):
            cps[cch].wait()
            pltpu.sync_copy(bufs[cch % 2],
                            sqkv_hbm.at[b, pl.ds(h * T + cch * GCH, GCH)])
            if cch + 2 < NCH:
                cps.append(fire(cch + 2))
        return _

    lax.fori_loop(0, TPW, task_body, 0)


def _sort_gather_call(buck, qkv_flat, nrm_flat):
    mesh = plsc.VectorSubcoreMesh(core_axis_name="c", subcore_axis_name="s")
    fn = functools.partial(
        pl.kernel,
        mesh=mesh,
        compiler_params=pltpu.CompilerParams(needs_layout_passes=False),
        out_type=[
            jax.ShapeDtypeStruct((B, NT // 128, 128), jnp.int32),
            jax.ShapeDtypeStruct((B, H, T // 128, 128), jnp.int32),
            jax.ShapeDtypeStruct((B, NT, DP), jnp.float32),
            jax.ShapeDtypeStruct((B, NT // 128, 128), jnp.float32),
        ],
        scratch_types=[
            pltpu.VMEM((T // 128, 128), jnp.int32),
            pltpu.VMEM((T,), jnp.int32),
            pltpu.VMEM((T // 128, 128), jnp.int32),
            pltpu.VMEM((T,), jnp.int32),
            pltpu.VMEM((T // 128, 128), jnp.int32),
            pltpu.VMEM((32,), jnp.int32),
            pltpu.VMEM((16,), jnp.int32),
            pltpu.VMEM((T // 128, 128), jnp.float32),
            pltpu.VMEM((T // 128, 128), jnp.float32),
            pltpu.VMEM((GCH, DP), jnp.float32),
            pltpu.VMEM((GCH, DP), jnp.float32),
            pltpu.SemaphoreType.DMA,
            pltpu.SemaphoreType.DMA,
        ],
    )(_sort_gather_kernel)
    return fn(buck, qkv_flat, nrm_flat)


ACH = 16
AR = ACH * BS


def _attn_body(qc, qp1, tkc, tkp, nkc, so_ref):
    mv = -jnp.finfo(jnp.float32).max
    qs = [qc[0, j * BS:(j + 1) * BS, :D] for j in range(ACH)]
    vs = [qc[0, j * BS:(j + 1) * BS, D:] for j in range(ACH)]
    qprev = qp1[0, :, :D]
    vprev = qp1[0, :, D:]
    dots_list = []
    for j in range(ACH):
        kmat = jnp.concatenate([qs[j], qprev if j == 0 else qs[j - 1]], axis=0)
        dots_list.append(
            lax.dot_general(qs[j], kmat, (((1,), (1,)), ((), ())),
                            preferred_element_type=jnp.float32))
    dots = jnp.concatenate(dots_list, axis=0)

    sel = (lax.broadcasted_iota(jnp.int32, (AR, BS), 1)
           == lax.broadcasted_iota(jnp.int32, (AR, BS), 0) % BS)
    tcur = tkc[0, :, 0]
    tcb = jnp.broadcast_to(tcur[:, None, :], (ACH, BS, BS)).reshape(AR, BS)
    qt = jnp.sum(jnp.where(sel, tcb, 0.0), axis=1, keepdims=True)
    ncur = nkc[0, :, 0]
    ncb = jnp.broadcast_to(ncur[:, None, :], (ACH, BS, BS)).reshape(AR, BS)
    nq = jnp.sum(jnp.where(sel, ncb, 0.0), axis=1, keepdims=True)
    dots = dots * (nq * 0.125)

    tshift = jnp.concatenate([tkp[0, 0], tcur[:-1]], axis=0)
    ktm = jnp.concatenate([tcur, tshift], axis=1)
    ktb = jnp.broadcast_to(ktm[:, None, :], (ACH, BS, 2 * BS)).reshape(AR, 2 * BS)
    dots = jnp.where(qt < ktb, mv, dots)
    dots = jnp.where(qt == ktb, SELF_VAL, dots)

    mx = jnp.max(dots, axis=1, keepdims=True)
    ex = jnp.exp(dots - mx)
    s = jnp.sum(ex, axis=1, keepdims=True)
    lse = jnp.log(s) + mx

    o_list = []
    for j in range(ACH):
        vmat = jnp.concatenate([vs[j], vprev if j == 0 else vs[j - 1]], axis=0)
        o_list.append(
            lax.dot_general(ex[j * BS:(j + 1) * BS], vmat,
                            (((1,), (0,)), ((), ())),
                            preferred_element_type=jnp.float32))
    ocat = jnp.concatenate(o_list, axis=0) / s
    so_ref[0] = jnp.concatenate(
        [ocat, jnp.broadcast_to(lse, (AR, D))], axis=1)


def _attn_call(sqkv, tk, nk):
    NI = NT // AR
    NB64 = NT // BS
    cur = pl.BlockSpec((1, AR, DP), lambda b, i: (b, i, 0))
    prev = pl.BlockSpec((1, BS, DP),
                        lambda b, i: (b, (i * ACH + NB64 - 1) % NB64, 0))
    tkc = pl.BlockSpec((1, ACH, 1, BS), lambda b, i: (b, i, 0, 0))
    tkp = pl.BlockSpec((1, 1, 1, BS),
                       lambda b, i: (b, (i * ACH + C - 1) % C, 0, 0))
    return pl.pallas_call(
        _attn_body,
        grid=(B, NI),
        in_specs=[cur, prev, tkc, tkp, tkc],
        out_specs=pl.BlockSpec((1, AR, DP), lambda b, i: (b, i, 0)),
        out_shape=jax.ShapeDtypeStruct((B, NT, DP), jnp.float32),
    )(sqkv, sqkv, tk, tk, nk)


def _unsort_kernel(pos_hbm, so_hbm,
                   ou_hbm,
                   pos_v, idxg_v, rows_a, rows_b, sem_a, sem_b):
    wid = lax.axis_index("c") * 16 + lax.axis_index("s")
    idx16 = lax.iota(jnp.int32, 16)

    def task_body(i, _):
        tid = wid * TPW + i
        b = tid // H
        h = tid % H
        pltpu.sync_copy(pos_hbm.at[b, h], pos_v)

        def blk(blk_i, _c):
            p = pos_v[blk_i // 8, pl.ds((blk_i % 8) * 16, 16)]
            idxg_v[pl.ds(blk_i * 16, 16)] = p + b * NT
            return _c

        lax.fori_loop(0, T // 16, blk, 0)
        NCH = T // GCH
        bufs = (rows_a, rows_b)
        sems = (sem_a, sem_b)

        def fire(cch):
            idx_sl = idxg_v.at[pl.ds(cch * GCH, GCH)]
            return pltpu.async_copy(so_hbm.at[idx_sl], bufs[cch % 2], sems[cch % 2])

        cps = [fire(0), fire(1)]
        for cch in range(NCH):
            cps[cch].wait()
            pltpu.sync_copy(bufs[cch % 2],
                            ou_hbm.at[b, h, pl.ds(cch * GCH, GCH)])
            if cch + 2 < NCH:
                cps.append(fire(cch + 2))
        return _

    lax.fori_loop(0, TPW, task_body, 0)


def _unsort_call(pos, so_flat):
    mesh = plsc.VectorSubcoreMesh(core_axis_name="c", subcore_axis_name="s")
    fn = functools.partial(
        pl.kernel,
        mesh=mesh,
        compiler_params=pltpu.CompilerParams(needs_layout_passes=False),
        out_type=jax.ShapeDtypeStruct((B, H, T, DP), jnp.float32),
        scratch_types=[
            pltpu.VMEM((T // 128, 128), jnp.int32),
            pltpu.VMEM((T,), jnp.int32),
            pltpu.VMEM((GCH, DP), jnp.float32),
            pltpu.VMEM((GCH, DP), jnp.float32),
            pltpu.SemaphoreType.DMA,
            pltpu.SemaphoreType.DMA,
        ],
    )(_unsort_kernel)
    return fn(pos, so_flat)


def _combine_body(o_ref, out_ref):
    l = o_ref[0, :, :, D:D + 1]
    mx = jnp.max(l, axis=0, keepdims=True)
    w = jnp.exp(l - mx)
    w = w / jnp.sum(w, axis=0, keepdims=True)
    acc = o_ref[0, 0, :, :D] * w[0]
    for h in range(1, H):
        acc = acc + o_ref[0, h, :, :D] * w[h]
    out_ref[0] = acc


def _combine_call(o_uns):
    return pl.pallas_call(
        _combine_body,
        grid=(B,),
        in_specs=[pl.BlockSpec((1, H, T, DP), lambda b: (b, 0, 0, 0))],
        out_specs=pl.BlockSpec((1, T, D), lambda b: (b, 0, 0)),
        out_shape=jax.ShapeDtypeStruct((B, T, D), jnp.float32),
    )(o_uns)


def kernel(qk, v, rotations):
    rot2 = rotations[0].reshape(D, H * 16)
    buck4, qkv, nrm = _hash_call(qk, v, rot2)
    st, pos, sqkv, nst = _sort_gather_call(buck4, qkv.reshape(B * T, DP), nrm)
    tk = st.astype(jnp.float32).reshape(B, C, 1, BS)
    nk = nst.reshape(B, C, 1, BS)
    so = _attn_call(sqkv, tk, nk)
    o_uns = _unsort_call(pos, so.reshape(B * NT, DP))
    return _combine_call(o_uns)


# speedup vs baseline: 1.6020x; 1.1158x over previous
import functools

import jax
import jax.numpy as jnp
from jax import lax
from jax.experimental import pallas as pl
from jax.experimental.pallas import tpu as pltpu
from jax.experimental.pallas import tpu_sc as plsc

B, T, D = 16, 2048, 64
H = 8
NBUCK = 32
BS = 64
C = H * NBUCK
NT = H * T
NW = 32
TPW = (B * H) // NW
SELF_VAL = -50000.0
GCH = 256
DP = 2 * D


def _hash_body(qk_ref, v_ref, rot_ref, buck_ref, qkv_ref, nrm_ref):
    x = qk_ref[0]
    rT = lax.dot_general(rot_ref[...], x, (((0,), (1,)), ((), ())),
                         preferred_element_type=jnp.float32)
    iota32 = lax.broadcasted_iota(jnp.int32, (NBUCK, T), 0)
    hrows = []
    for h in range(H):
        sub = rT[h * 16:(h + 1) * 16]
        seg = jnp.concatenate([sub, -sub], axis=0)
        m = jnp.max(seg, axis=0, keepdims=True)
        am = jnp.min(jnp.where(seg == m, iota32, NBUCK), axis=0, keepdims=True)
        blocks = [am[:, k * 128:(k + 1) * 128] for k in range(T // 128)]
        hrows.append(jnp.concatenate(blocks, axis=0).reshape(1, T // 128, 128))
    buck_ref[0] = jnp.concatenate(hrows, axis=0)
    xsq = x * x
    n_row = jnp.sum(xsq, axis=1, keepdims=True)
    qkn = x * (1.0 / jnp.maximum(jnp.sqrt(n_row), 1e-12))
    qkv_ref[0] = jnp.concatenate([qkn, v_ref[0]], axis=1)
    n_lane = jnp.sqrt(lax.dot_general(
        jnp.ones((1, D), jnp.float32), xsq, (((1,), (1,)), ((), ())),
        preferred_element_type=jnp.float32))
    nbl = [n_lane[:, k * 128:(k + 1) * 128] for k in range(T // 128)]
    nrm_ref[0] = jnp.concatenate(nbl, axis=0)


def _hash_call(qk, v, rot2):
    return pl.pallas_call(
        _hash_body,
        grid=(B,),
        in_specs=[
            pl.BlockSpec((1, T, D), lambda b: (b, 0, 0)),
            pl.BlockSpec((1, T, D), lambda b: (b, 0, 0)),
            pl.BlockSpec((D, H * 16), lambda b: (0, 0)),
        ],
        out_specs=[
            pl.BlockSpec((1, H, T // 128, 128), lambda b: (b, 0, 0, 0)),
            pl.BlockSpec((1, T, DP), lambda b: (b, 0, 0)),
            pl.BlockSpec((1, T // 128, 128), lambda b: (b, 0, 0)),
        ],
        out_shape=[
            jax.ShapeDtypeStruct((B, H, T // 128, 128), jnp.int32),
            jax.ShapeDtypeStruct((B, T, DP), jnp.float32),
            jax.ShapeDtypeStruct((B, T // 128, 128), jnp.float32),
        ],
    )(qk, v, rot2)


def _sort_gather_kernel(buck_hbm, qkv_hbm, nrm_hbm,
                        st_hbm, pos_hbm, sqkv_hbm, nst_hbm,
                        buk_v, rank_v, st_v, idxg_v, pos_v,
                        cnt_v, bs_v, nrm_v, nst_v, rows_a, rows_b, sem_a, sem_b):
    wid = lax.axis_index("c") * 16 + lax.axis_index("s")
    idx16 = lax.iota(jnp.int32, 16)
    zeros16 = jnp.zeros((16,), jnp.int32)

    def task_body(i, _):
        tid = wid * TPW + i
        b = tid // H
        h = tid % H
        pltpu.sync_copy(buck_hbm.at[b, h], buk_v)
        pltpu.sync_copy(nrm_hbm.at[b], nrm_v)
        cnt_v[pl.ds(0, 16)] = zeros16
        cnt_v[pl.ds(16, 16)] = zeros16

        def blk1(blk, _c):
            bvec = buk_v[blk // 8, pl.ds((blk % 8) * 16, 16)]
            key = bvec * 16 + idx16
            ks, vs = plsc.sort_key_val(key, idx16)
            bs = lax.shift_right_logical(ks, 4)
            bs_v[...] = bs
            prev = plsc.load_gather(bs_v, [jnp.maximum(idx16 - 1, 0)])
            is_start = (idx16 == 0) | (bs != prev)
            start_idx = plsc.cummax(jnp.where(is_start, idx16, 0))
            rnk = (idx16 - start_idx) + plsc.load_gather(cnt_v, [bs])
            nxt = plsc.load_gather(bs_v, [jnp.minimum(idx16 + 1, 15)])
            is_end = (idx16 == 15) | (bs != nxt)
            plsc.store_scatter(cnt_v, [bs], rnk + 1, mask=is_end)
            plsc.store_scatter(rank_v, [blk * 16 + vs], rnk)
            return _c

        lax.fori_loop(0, T // 16, blk1, 0)

        c0 = cnt_v[pl.ds(0, 16)]
        c1 = cnt_v[pl.ds(16, 16)]
        t0 = jnp.sum(c0)
        base0 = plsc.cumsum(c0) - c0
        base1 = plsc.cumsum(c1) + t0 - c1
        cnt_v[pl.ds(0, 16)] = base0
        cnt_v[pl.ds(16, 16)] = base1

        def blk2(blk, _c):
            r = blk // 8
            csl = pl.ds((blk % 8) * 16, 16)
            bvec = buk_v[r, csl]
            rnk = rank_v[pl.ds(blk * 16, 16)]
            ploc = plsc.load_gather(cnt_v, [bvec]) + rnk
            pr = lax.shift_right_logical(ploc, 7)
            pc = lax.bitwise_and(ploc, 127)
            tvec = blk * 16 + idx16
            pos_v[r, csl] = ploc + h * T
            plsc.store_scatter(st_v, [pr, pc], tvec)
            plsc.store_scatter(idxg_v, [ploc], tvec + b * T)
            plsc.store_scatter(nst_v, [pr, pc], nrm_v[r, csl])
            return _c

        lax.fori_loop(0, T // 16, blk2, 0)

        pltpu.sync_copy(pos_v, pos_hbm.at[b, h])
        pltpu.sync_copy(st_v, st_hbm.at[b, pl.ds(h * (T // 128), T // 128)])
        pltpu.sync_copy(nst_v, nst_hbm.at[b, pl.ds(h * (T // 128), T // 128)])

        NCH = T // GCH
        bufs = (rows_a, rows_b)
        sems = (sem_a, sem_b)

        def fire(cch):
            idx_sl = idxg_v.at[pl.ds(cch * GCH, GCH)]
            return pltpu.async_copy(qkv_hbm.at[idx_sl], bufs[cch % 2], sems[cch % 2])

        cps = [fire(0), fire(1)]
        for cch in range(NCH):
            cps[cch].wait()
            pltpu.sync_copy(bufs[cch % 2],
                            sqkv_hbm.at[b, pl.ds(h * T + cch * GCH, GCH)])
            if cch + 2 < NCH:
                cps.append(fire(cch + 2))
        return _

    lax.fori_loop(0, TPW, task_body, 0)


def _sort_gather_call(buck, qkv_flat, nrm_flat):
    mesh = plsc.VectorSubcoreMesh(core_axis_name="c", subcore_axis_name="s")
    fn = functools.partial(
        pl.kernel,
        mesh=mesh,
        compiler_params=pltpu.CompilerParams(needs_layout_passes=False),
        out_type=[
            jax.ShapeDtypeStruct((B, NT // 128, 128), jnp.int32),
            jax.ShapeDtypeStruct((B, H, T // 128, 128), jnp.int32),
            jax.ShapeDtypeStruct((B, NT, DP), jnp.float32),
            jax.ShapeDtypeStruct((B, NT // 128, 128), jnp.float32),
        ],
        scratch_types=[
            pltpu.VMEM((T // 128, 128), jnp.int32),
            pltpu.VMEM((T,), jnp.int32),
            pltpu.VMEM((T // 128, 128), jnp.int32),
            pltpu.VMEM((T,), jnp.int32),
            pltpu.VMEM((T // 128, 128), jnp.int32),
            pltpu.VMEM((32,), jnp.int32),
            pltpu.VMEM((16,), jnp.int32),
            pltpu.VMEM((T // 128, 128), jnp.float32),
            pltpu.VMEM((T // 128, 128), jnp.float32),
            pltpu.VMEM((GCH, DP), jnp.float32),
            pltpu.VMEM((GCH, DP), jnp.float32),
            pltpu.SemaphoreType.DMA,
            pltpu.SemaphoreType.DMA,
        ],
    )(_sort_gather_kernel)
    return fn(buck, qkv_flat, nrm_flat)


ACH = 32
AR = ACH * BS


def _attn_body(qc, qp1, tkc, tkp, nkc, so_ref):
    mv = -jnp.finfo(jnp.float32).max
    qs = [qc[0, j * BS:(j + 1) * BS, :D] for j in range(ACH)]
    vs = [qc[0, j * BS:(j + 1) * BS, D:] for j in range(ACH)]
    qprev = qp1[0, :, :D]
    vprev = qp1[0, :, D:]
    dots_list = []
    for j in range(ACH):
        kmat = jnp.concatenate([qs[j], qprev if j == 0 else qs[j - 1]], axis=0)
        dots_list.append(
            lax.dot_general(qs[j], kmat, (((1,), (1,)), ((), ())),
                            preferred_element_type=jnp.float32))
    dots = jnp.concatenate(dots_list, axis=0)

    sel = (lax.broadcasted_iota(jnp.int32, (AR, BS), 1)
           == lax.broadcasted_iota(jnp.int32, (AR, BS), 0) % BS)
    tcur = tkc[0, :, 0]
    tcb = jnp.broadcast_to(tcur[:, None, :], (ACH, BS, BS)).reshape(AR, BS)
    qt = jnp.sum(jnp.where(sel, tcb, 0.0), axis=1, keepdims=True)
    ncur = nkc[0, :, 0]
    ncb = jnp.broadcast_to(ncur[:, None, :], (ACH, BS, BS)).reshape(AR, BS)
    nq = jnp.sum(jnp.where(sel, ncb, 0.0), axis=1, keepdims=True)
    dots = dots * (nq * 0.125)

    tshift = jnp.concatenate([tkp[0, 0], tcur[:-1]], axis=0)
    ktm = jnp.concatenate([tcur, tshift], axis=1)
    ktb = jnp.broadcast_to(ktm[:, None, :], (ACH, BS, 2 * BS)).reshape(AR, 2 * BS)
    dots = jnp.where(qt < ktb, mv, dots)
    dots = jnp.where(qt == ktb, SELF_VAL, dots)

    mx = jnp.max(dots, axis=1, keepdims=True)
    ex = jnp.exp(dots - mx)
    s = jnp.sum(ex, axis=1, keepdims=True)
    lse = jnp.log(s) + mx

    o_list = []
    for j in range(ACH):
        vmat = jnp.concatenate([vs[j], vprev if j == 0 else vs[j - 1]], axis=0)
        o_list.append(
            lax.dot_general(ex[j * BS:(j + 1) * BS], vmat,
                            (((1,), (0,)), ((), ())),
                            preferred_element_type=jnp.float32))
    ocat = jnp.concatenate(o_list, axis=0) / s
    so_ref[0] = jnp.concatenate(
        [ocat, jnp.broadcast_to(lse, (AR, D))], axis=1)


def _attn_call(sqkv, tk, nk):
    NI = NT // AR
    NB64 = NT // BS
    cur = pl.BlockSpec((1, AR, DP), lambda b, i: (b, i, 0))
    prev = pl.BlockSpec((1, BS, DP),
                        lambda b, i: (b, (i * ACH + NB64 - 1) % NB64, 0))
    tkc = pl.BlockSpec((1, ACH, 1, BS), lambda b, i: (b, i, 0, 0))
    tkp = pl.BlockSpec((1, 1, 1, BS),
                       lambda b, i: (b, (i * ACH + C - 1) % C, 0, 0))
    return pl.pallas_call(
        _attn_body,
        grid=(B, NI),
        in_specs=[cur, prev, tkc, tkp, tkc],
        out_specs=pl.BlockSpec((1, AR, DP), lambda b, i: (b, i, 0)),
        out_shape=jax.ShapeDtypeStruct((B, NT, DP), jnp.float32),
    )(sqkv, sqkv, tk, tk, nk)


def _unsort_kernel(pos_hbm, so_hbm,
                   ou_hbm,
                   pos_v, idxg_v, rows_a, rows_b, sem_a, sem_b):
    wid = lax.axis_index("c") * 16 + lax.axis_index("s")
    idx16 = lax.iota(jnp.int32, 16)

    def task_body(i, _):
        tid = wid * TPW + i
        b = tid // H
        h = tid % H
        pltpu.sync_copy(pos_hbm.at[b, h], pos_v)

        def blk(blk_i, _c):
            p = pos_v[blk_i // 8, pl.ds((blk_i % 8) * 16, 16)]
            idxg_v[pl.ds(blk_i * 16, 16)] = p + b * NT
            return _c

        lax.fori_loop(0, T // 16, blk, 0)
        NCH = T // GCH
        bufs = (rows_a, rows_b)
        sems = (sem_a, sem_b)

        def fire(cch):
            idx_sl = idxg_v.at[pl.ds(cch * GCH, GCH)]
            return pltpu.async_copy(so_hbm.at[idx_sl], bufs[cch % 2], sems[cch % 2])

        cps = [fire(0), fire(1)]
        for cch in range(NCH):
            cps[cch].wait()
            pltpu.sync_copy(bufs[cch % 2],
                            ou_hbm.at[b, h, pl.ds(cch * GCH, GCH)])
            if cch + 2 < NCH:
                cps.append(fire(cch + 2))
        return _

    lax.fori_loop(0, TPW, task_body, 0)


def _unsort_call(pos, so_flat):
    mesh = plsc.VectorSubcoreMesh(core_axis_name="c", subcore_axis_name="s")
    fn = functools.partial(
        pl.kernel,
        mesh=mesh,
        compiler_params=pltpu.CompilerParams(needs_layout_passes=False),
        out_type=jax.ShapeDtypeStruct((B, H, T, DP), jnp.float32),
        scratch_types=[
            pltpu.VMEM((T // 128, 128), jnp.int32),
            pltpu.VMEM((T,), jnp.int32),
            pltpu.VMEM((GCH, DP), jnp.float32),
            pltpu.VMEM((GCH, DP), jnp.float32),
            pltpu.SemaphoreType.DMA,
            pltpu.SemaphoreType.DMA,
        ],
    )(_unsort_kernel)
    return fn(pos, so_flat)


def _combine_body(o_ref, out_ref):
    l = o_ref[0, :, :, D:D + 1]
    mx = jnp.max(l, axis=0, keepdims=True)
    w = jnp.exp(l - mx)
    w = w / jnp.sum(w, axis=0, keepdims=True)
    acc = o_ref[0, 0, :, :D] * w[0]
    for h in range(1, H):
        acc = acc + o_ref[0, h, :, :D] * w[h]
    out_ref[0] = acc


def _combine_call(o_uns):
    return pl.pallas_call(
        _combine_body,
        grid=(B,),
        in_specs=[pl.BlockSpec((1, H, T, DP), lambda b: (b, 0, 0, 0))],
        out_specs=pl.BlockSpec((1, T, D), lambda b: (b, 0, 0)),
        out_shape=jax.ShapeDtypeStruct((B, T, D), jnp.float32),
    )(o_uns)


def kernel(qk, v, rotations):
    rot2 = rotations[0].reshape(D, H * 16)
    buck4, qkv, nrm = _hash_call(qk, v, rot2)
    st, pos, sqkv, nst = _sort_gather_call(buck4, qkv.reshape(B * T, DP), nrm)
    tk = st.astype(jnp.float32).reshape(B, C, 1, BS)
    nk = nst.reshape(B, C, 1, BS)
    so = _attn_call(sqkv, tk, nk)
    o_uns = _unsort_call(pos, so.reshape(B * NT, DP))
    return _combine_call(o_uns)


# speedup vs baseline: 1.8853x; 1.1769x over previous
import functools

import jax
import jax.numpy as jnp
from jax import lax
from jax.experimental import pallas as pl
from jax.experimental.pallas import tpu as pltpu
from jax.experimental.pallas import tpu_sc as plsc

B, T, D = 16, 2048, 64
H = 8
NBUCK = 32
BS = 64
C = H * NBUCK
NT = H * T
NW = 32
TPW = (B * H) // NW
SELF_VAL = -50000.0
GCH = 256
DP = 2 * D


def _hash_body(qk_ref, v_ref, rot_ref, buck_ref, qkv_ref, nrm_ref):
    x = qk_ref[0]
    rT = lax.dot_general(rot_ref[...], x, (((0,), (1,)), ((), ())),
                         preferred_element_type=jnp.float32)
    iota32 = lax.broadcasted_iota(jnp.int32, (NBUCK, T), 0)
    hrows = []
    for h in range(H):
        sub = rT[h * 16:(h + 1) * 16]
        seg = jnp.concatenate([sub, -sub], axis=0)
        m = jnp.max(seg, axis=0, keepdims=True)
        am = jnp.min(jnp.where(seg == m, iota32, NBUCK), axis=0, keepdims=True)
        blocks = [am[:, k * 128:(k + 1) * 128] for k in range(T // 128)]
        hrows.append(jnp.concatenate(blocks, axis=0).reshape(1, T // 128, 128))
    buck_ref[0] = jnp.concatenate(hrows, axis=0)
    xsq = x * x
    n_row = jnp.sum(xsq, axis=1, keepdims=True)
    qkn = x * (1.0 / jnp.maximum(jnp.sqrt(n_row), 1e-12))
    qkv_ref[0] = jnp.concatenate([qkn, v_ref[0]], axis=1)
    n_lane = jnp.sqrt(lax.dot_general(
        jnp.ones((1, D), jnp.float32), xsq, (((1,), (1,)), ((), ())),
        preferred_element_type=jnp.float32))
    nbl = [n_lane[:, k * 128:(k + 1) * 128] for k in range(T // 128)]
    nrm_ref[0] = jnp.concatenate(nbl, axis=0)


def _hash_call(qk, v, rot2):
    return pl.pallas_call(
        _hash_body,
        grid=(B,),
        in_specs=[
            pl.BlockSpec((1, T, D), lambda b: (b, 0, 0)),
            pl.BlockSpec((1, T, D), lambda b: (b, 0, 0)),
            pl.BlockSpec((D, H * 16), lambda b: (0, 0)),
        ],
        out_specs=[
            pl.BlockSpec((1, H, T // 128, 128), lambda b: (b, 0, 0, 0)),
            pl.BlockSpec((1, T, DP), lambda b: (b, 0, 0)),
            pl.BlockSpec((1, T // 128, 128), lambda b: (b, 0, 0)),
        ],
        out_shape=[
            jax.ShapeDtypeStruct((B, H, T // 128, 128), jnp.int32),
            jax.ShapeDtypeStruct((B, T, DP), jnp.float32),
            jax.ShapeDtypeStruct((B, T // 128, 128), jnp.float32),
        ],
    )(qk, v, rot2)


def _sort_gather_kernel(buck_hbm, qkv_hbm, nrm_hbm,
                        st_hbm, pos_hbm, sqkv_hbm, nst_hbm,
                        buk_v, rank_v, st_v, idxg_v, pos_v,
                        cnt_v, bs_v, nrm_v, nst_v, rows_a, rows_b, sem_a, sem_b):
    wid = lax.axis_index("c") * 16 + lax.axis_index("s")
    idx16 = lax.iota(jnp.int32, 16)
    zeros16 = jnp.zeros((16,), jnp.int32)

    def task_body(i, _):
        tid = wid * TPW + i
        b = tid // H
        h = tid % H
        pltpu.sync_copy(buck_hbm.at[b, h], buk_v)
        pltpu.sync_copy(nrm_hbm.at[b], nrm_v)
        cnt_v[pl.ds(0, 16)] = zeros16
        cnt_v[pl.ds(16, 16)] = zeros16

        def blk1(blk, _c):
            bvec = buk_v[blk // 8, pl.ds((blk % 8) * 16, 16)]
            key = bvec * 16 + idx16
            ks, vs = plsc.sort_key_val(key, idx16)
            bs = lax.shift_right_logical(ks, 4)
            bs_v[...] = bs
            prev = plsc.load_gather(bs_v, [jnp.maximum(idx16 - 1, 0)])
            is_start = (idx16 == 0) | (bs != prev)
            start_idx = plsc.cummax(jnp.where(is_start, idx16, 0))
            rnk = (idx16 - start_idx) + plsc.load_gather(cnt_v, [bs])
            nxt = plsc.load_gather(bs_v, [jnp.minimum(idx16 + 1, 15)])
            is_end = (idx16 == 15) | (bs != nxt)
            plsc.store_scatter(cnt_v, [bs], rnk + 1, mask=is_end)
            plsc.store_scatter(rank_v, [blk * 16 + vs], rnk)
            return _c

        lax.fori_loop(0, T // 16, blk1, 0)

        c0 = cnt_v[pl.ds(0, 16)]
        c1 = cnt_v[pl.ds(16, 16)]
        t0 = jnp.sum(c0)
        base0 = plsc.cumsum(c0) - c0
        base1 = plsc.cumsum(c1) + t0 - c1
        cnt_v[pl.ds(0, 16)] = base0
        cnt_v[pl.ds(16, 16)] = base1

        def blk2(blk, _c):
            r = blk // 8
            csl = pl.ds((blk % 8) * 16, 16)
            bvec = buk_v[r, csl]
            rnk = rank_v[pl.ds(blk * 16, 16)]
            ploc = plsc.load_gather(cnt_v, [bvec]) + rnk
            pr = lax.shift_right_logical(ploc, 7)
            pc = lax.bitwise_and(ploc, 127)
            tvec = blk * 16 + idx16
            pos_v[r, csl] = ploc + h * T
            plsc.store_scatter(st_v, [pr, pc], tvec)
            plsc.store_scatter(idxg_v, [ploc], tvec + b * T)
            plsc.store_scatter(nst_v, [pr, pc], nrm_v[r, csl])
            return _c

        lax.fori_loop(0, T // 16, blk2, 0)

        pltpu.sync_copy(pos_v, pos_hbm.at[b, h])
        pltpu.sync_copy(st_v, st_hbm.at[b, pl.ds(h * (T // 128), T // 128)])
        pltpu.sync_copy(nst_v, nst_hbm.at[b, pl.ds(h * (T // 128), T // 128)])

        NCH = T // GCH
        bufs = (rows_a, rows_b)
        sems = (sem_a, sem_b)

        def fire(cch):
            idx_sl = idxg_v.at[pl.ds(cch * GCH, GCH)]
            return pltpu.async_copy(qkv_hbm.at[idx_sl], bufs[cch % 2], sems[cch % 2])

        cps = [fire(0), fire(1)]
        for cch in range(NCH):
            cps[cch].wait()
            pltpu.sync_copy(bufs[cch % 2],
                            sqkv_hbm.at[b, pl.ds(h * T + cch * GCH, GCH)])
            if cch + 2 < NCH:
                cps.append(fire(cch + 2))
        return _

    lax.fori_loop(0, TPW, task_body, 0)


def _sort_gather_call(buck, qkv_flat, nrm_flat):
    mesh = plsc.VectorSubcoreMesh(core_axis_name="c", subcore_axis_name="s")
    fn = functools.partial(
        pl.kernel,
        mesh=mesh,
        compiler_params=pltpu.CompilerParams(needs_layout_passes=False),
        out_type=[
            jax.ShapeDtypeStruct((B, NT // 128, 128), jnp.int32),
            jax.ShapeDtypeStruct((B, H, T // 128, 128), jnp.int32),
            jax.ShapeDtypeStruct((B, NT, DP), jnp.float32),
            jax.ShapeDtypeStruct((B, NT // 128, 128), jnp.float32),
        ],
        scratch_types=[
            pltpu.VMEM((T // 128, 128), jnp.int32),
            pltpu.VMEM((T,), jnp.int32),
            pltpu.VMEM((T // 128, 128), jnp.int32),
            pltpu.VMEM((T,), jnp.int32),
            pltpu.VMEM((T // 128, 128), jnp.int32),
            pltpu.VMEM((32,), jnp.int32),
            pltpu.VMEM((16,), jnp.int32),
            pltpu.VMEM((T // 128, 128), jnp.float32),
            pltpu.VMEM((T // 128, 128), jnp.float32),
            pltpu.VMEM((GCH, DP), jnp.float32),
            pltpu.VMEM((GCH, DP), jnp.float32),
            pltpu.SemaphoreType.DMA,
            pltpu.SemaphoreType.DMA,
        ],
    )(_sort_gather_kernel)
    return fn(buck, qkv_flat, nrm_flat)


ACH = 32
AR = ACH * BS


def _attn_body(qc, qp1, tkc, tkp, nkc, so_ref):
    mv = -jnp.finfo(jnp.float32).max
    qs = [qc[0, j * BS:(j + 1) * BS, :D] for j in range(ACH)]
    vs = [qc[0, j * BS:(j + 1) * BS, D:] for j in range(ACH)]
    qprev = qp1[0, :, :D]
    vprev = qp1[0, :, D:]
    dots_list = []
    for j in range(ACH):
        kmat = jnp.concatenate([qs[j], qprev if j == 0 else qs[j - 1]], axis=0)
        dots_list.append(
            lax.dot_general(qs[j], kmat, (((1,), (1,)), ((), ())),
                            preferred_element_type=jnp.float32))
    dots = jnp.concatenate(dots_list, axis=0)

    sel = (lax.broadcasted_iota(jnp.int32, (AR, BS), 1)
           == lax.broadcasted_iota(jnp.int32, (AR, BS), 0) % BS)
    tcur = tkc[0, :, 0]
    tcb = jnp.broadcast_to(tcur[:, None, :], (ACH, BS, BS)).reshape(AR, BS)
    qt = jnp.sum(jnp.where(sel, tcb, 0.0), axis=1, keepdims=True)
    ncur = nkc[0, :, 0]
    ncb = jnp.broadcast_to(ncur[:, None, :], (ACH, BS, BS)).reshape(AR, BS)
    nq = jnp.sum(jnp.where(sel, ncb, 0.0), axis=1, keepdims=True)
    dots = dots * (nq * 0.125)

    tshift = jnp.concatenate([tkp[0, 0], tcur[:-1]], axis=0)
    ktm = jnp.concatenate([tcur, tshift], axis=1)
    ktb = jnp.broadcast_to(ktm[:, None, :], (ACH, BS, 2 * BS)).reshape(AR, 2 * BS)
    dots = jnp.where(qt < ktb, mv, dots)
    dots = jnp.where(qt == ktb, SELF_VAL, dots)

    mx = jnp.max(dots, axis=1, keepdims=True)
    ex = jnp.exp(dots - mx)
    s = jnp.sum(ex, axis=1, keepdims=True)
    lse = jnp.log(s) + mx

    o_list = []
    for j in range(ACH):
        vmat = jnp.concatenate([vs[j], vprev if j == 0 else vs[j - 1]], axis=0)
        o_list.append(
            lax.dot_general(ex[j * BS:(j + 1) * BS], vmat,
                            (((1,), (0,)), ((), ())),
                            preferred_element_type=jnp.float32))
    ocat = jnp.concatenate(o_list, axis=0) / s
    so_ref[0] = jnp.concatenate(
        [ocat, jnp.broadcast_to(lse, (AR, D))], axis=1)


def _attn_call(sqkv, tk, nk):
    NI = NT // AR
    NB64 = NT // BS
    cur = pl.BlockSpec((1, AR, DP), lambda b, i: (b, i, 0))
    prev = pl.BlockSpec((1, BS, DP),
                        lambda b, i: (b, (i * ACH + NB64 - 1) % NB64, 0))
    tkc = pl.BlockSpec((1, ACH, 1, BS), lambda b, i: (b, i, 0, 0))
    tkp = pl.BlockSpec((1, 1, 1, BS),
                       lambda b, i: (b, (i * ACH + C - 1) % C, 0, 0))
    return pl.pallas_call(
        _attn_body,
        grid=(B, NI),
        in_specs=[cur, prev, tkc, tkp, tkc],
        out_specs=pl.BlockSpec((1, AR, DP), lambda b, i: (b, i, 0)),
        out_shape=jax.ShapeDtypeStruct((B, NT, DP), jnp.float32),
    )(sqkv, sqkv, tk, tk, nk)


TG = 32
NG = (T // 2) // TG


def _unsort_combine_kernel(pos_hbm, so_hbm,
                           out_hbm,
                           pos_v, idx_a, idx_b, buf_a, buf_b,
                           w_v, obuf, sem_a, sem_b):
    wid = lax.axis_index("c") * 16 + lax.axis_index("s")
    b = wid // 2
    half = wid % 2
    idx16 = lax.iota(jnp.int32, 16)
    lane8 = lax.bitwise_and(idx16, 7)
    col_lse = lax.bitwise_and(idx16, 0) + D
    for h in range(H):
        pltpu.sync_copy(pos_hbm.at[b, h, pl.ds(pl.multiple_of(half * 8, 8), 8)],
                        pos_v.at[h])

    idxs = (idx_a, idx_b)
    bufs = (buf_a, buf_b)
    sems = (sem_a, sem_b)

    def build_and_fire(g, par):
        r = g // 4
        lb = (g % 4) * 32
        idxb = idxs[par]
        for h in range(H):
            for s in range(2):
                p16 = pos_v[h, r, pl.ds(lb + s * 16, 16)]
                plsc.store_scatter(idxb, [(s * 16 + idx16) * H + h], p16 + b * NT)
        pltpu.async_copy(so_hbm.at[idxb], bufs[par], sems[par])

    build_and_fire(0, 0)
    build_and_fire(1, 1)

    def pair_body(gp, _):
        for par in range(2):
            g = gp * 2 + par
            buf = bufs[par]
            pltpu.make_async_copy(so_hbm.at[pl.ds(0, TG * H)], buf, sems[par]).wait()

            def tok_body(j, _c):
                base = j * H
                ls = [buf[base + h, pl.ds(D, 16)] for h in range(H)]
                m = ls[0]
                for h in range(1, H):
                    m = jnp.maximum(m, ls[h])
                es = [jnp.exp(l - m) for l in ls]
                ssum = es[0]
                for h in range(1, H):
                    ssum = ssum + es[h]
                inv = 1.0 / ssum
                accs = [None] * 4
                for h in range(H):
                    wb = es[h] * inv
                    row = base + h
                    for cc in range(4):
                        part = wb * buf[row, pl.ds(cc * 16, 16)]
                        accs[cc] = part if h == 0 else accs[cc] + part
                for cc in range(4):
                    obuf[j // 2, pl.ds((j % 2) * 64 + cc * 16, 16)] = accs[cc]
                return _c

            lax.fori_loop(0, TG, tok_body, 0)
            orow = pl.multiple_of((half * (T // 2) + g * TG) // 2, TG // 2)
            pltpu.sync_copy(obuf, out_hbm.at[b, pl.ds(orow, TG // 2)])

            @pl.when(g + 2 < NG)
            def _fire_next():
                build_and_fire(g + 2, par)
        return _

    lax.fori_loop(0, NG // 2, pair_body, 0)


def _unsort_combine_call(pos, so_flat):
    mesh = plsc.VectorSubcoreMesh(core_axis_name="c", subcore_axis_name="s")
    fn = functools.partial(
        pl.kernel,
        mesh=mesh,
        compiler_params=pltpu.CompilerParams(needs_layout_passes=False),
        out_type=jax.ShapeDtypeStruct((B, T * D // 128, 128), jnp.float32),
        scratch_types=[
            pltpu.VMEM((H, 8, 128), jnp.int32),
            pltpu.VMEM((TG * H,), jnp.int32),
            pltpu.VMEM((TG * H,), jnp.int32),
            pltpu.VMEM((TG * H, DP), jnp.float32),
            pltpu.VMEM((TG * H, DP), jnp.float32),
            pltpu.VMEM((16,), jnp.float32),
            pltpu.VMEM((TG * D // 128, 128), jnp.float32),
            pltpu.SemaphoreType.DMA,
            pltpu.SemaphoreType.DMA,
        ],
    )(_unsort_combine_kernel)
    return fn(pos, so_flat)


def kernel(qk, v, rotations):
    rot2 = rotations[0].reshape(D, H * 16)
    buck4, qkv, nrm = _hash_call(qk, v, rot2)
    st, pos, sqkv, nst = _sort_gather_call(buck4, qkv.reshape(B * T, DP), nrm)
    tk = st.astype(jnp.float32).reshape(B, C, 1, BS)
    nk = nst.reshape(B, C, 1, BS)
    so = _attn_call(sqkv, tk, nk)
    out = _unsort_combine_call(pos, so.reshape(B * NT, DP))
    return out.reshape(B, T, D)
